# Initial kernel scaffold; baseline (speedup 1.0000x reference)
import jax
import jax.numpy as jnp
from jax.experimental import pallas as pl


def kernel(graph_or_x, edge_index):
    raise NotImplementedError("write your pallas kernel here")


# speedup vs baseline: 2.6129x; 2.6129x over previous
import functools

import jax
import jax.numpy as jnp
from jax import lax
from jax.experimental import pallas as pl
from jax.experimental.pallas import tpu as pltpu
from jax.experimental.pallas import tpu_sc as plsc

_N = 10000
_E = 320000
_D = 128
_STEPS = 10
_ALPHA = 0.1

_NC = 2
_NS = 16
_NW = _NC * _NS
_CHUNK = 128
_NPAD = 10240
_ROWS_PER_TILE = _NPAD // _NS
_EPW = 10240
_NCH = _EPW // _CHUNK
_EPAD = _NW * _EPW

_mesh = plsc.VectorSubcoreMesh(core_axis_name="c", subcore_axis_name="s")


def _acc_body(cur_hbm, srcr, dstr, zeros_hbm, part_hbm,
              src_v, dst_v, rows, sem, acc_sh):
    c = lax.axis_index("c")
    s = lax.axis_index("s")
    wid = c * _NS + s
    pltpu.sync_copy(srcr.at[wid], src_v)
    pltpu.sync_copy(dstr.at[wid], dst_v)
    pltpu.sync_copy(zeros_hbm, acc_sh.at[pl.ds(s * _ROWS_PER_TILE, _ROWS_PER_TILE)])
    plsc.subcore_barrier()

    def body(j, carry):
        pltpu.async_copy(cur_hbm.at[src_v.at[j]], rows, sem).wait()
        pltpu.sync_copy(rows, acc_sh.at[dst_v.at[j]], add=True)
        return carry

    lax.fori_loop(0, _NCH, body, None)
    plsc.subcore_barrier()
    base = c * _NPAD + s * _ROWS_PER_TILE
    pltpu.sync_copy(acc_sh.at[pl.ds(s * _ROWS_PER_TILE, _ROWS_PER_TILE)],
                    part_hbm.at[pl.ds(base, _ROWS_PER_TILE)])


_acc_kernel = functools.partial(
    pl.kernel,
    out_type=jax.ShapeDtypeStruct((_NC * _NPAD, _D), jnp.float32),
    mesh=_mesh,
    scratch_types=[
        pltpu.VMEM((_NCH, _CHUNK), jnp.int32),
        pltpu.VMEM((_NCH, _CHUNK), jnp.int32),
        pltpu.VMEM((_CHUNK, _D), jnp.float32),
        pltpu.SemaphoreType.DMA,
        pltpu.VMEM_SHARED((_NPAD, _D), jnp.float32),
    ],
)(_acc_body)


def _deg_body(dstr, ones_hbm, zeros_hbm, part_hbm,
              dst_v, ones_v, acc_sh):
    c = lax.axis_index("c")
    s = lax.axis_index("s")
    wid = c * _NS + s
    pltpu.sync_copy(dstr.at[wid], dst_v)
    pltpu.sync_copy(ones_hbm, ones_v)
    pltpu.sync_copy(zeros_hbm, acc_sh.at[pl.ds(s * _ROWS_PER_TILE, _ROWS_PER_TILE)])
    plsc.subcore_barrier()

    def body(j, carry):
        pltpu.sync_copy(ones_v, acc_sh.at[dst_v.at[j]], add=True)
        return carry

    lax.fori_loop(0, _NCH, body, None)
    plsc.subcore_barrier()
    base = c * _NPAD + s * _ROWS_PER_TILE
    pltpu.sync_copy(acc_sh.at[pl.ds(s * _ROWS_PER_TILE, _ROWS_PER_TILE)],
                    part_hbm.at[pl.ds(base, _ROWS_PER_TILE)])


_deg_kernel = functools.partial(
    pl.kernel,
    out_type=jax.ShapeDtypeStruct((_NC * _NPAD, _D), jnp.float32),
    mesh=_mesh,
    scratch_types=[
        pltpu.VMEM((_NCH, _CHUNK), jnp.int32),
        pltpu.VMEM((_CHUNK, _D), jnp.float32),
        pltpu.VMEM_SHARED((_NPAD, _D), jnp.float32),
    ],
)(_deg_body)


_BLK = 1280
_G = _NPAD // _BLK


def _p0_spec():
    return pl.BlockSpec((_BLK, _D), lambda i: (i, 0))


def _p1_spec():
    return pl.BlockSpec((_BLK, _D), lambda i: (i + _G, 0))


def _inv_body(d0, d1, inv):
    inv[...] = 1.0 / jnp.maximum(d0[...] + d1[...], 1.0)


_inv_kernel = pl.pallas_call(
    _inv_body,
    grid=(_G,),
    in_specs=[_p0_spec(), _p1_spec()],
    out_specs=_p0_spec(),
    out_shape=jax.ShapeDtypeStruct((_NPAD, _D), jnp.float32),
)


def _comb_body(p0, p1, inv, osum, nxt, osum_o):
    nv = (p0[...] + p1[...]) * inv[...]
    nxt[...] = nv
    osum_o[...] = osum[...] + nv


_comb_kernel = pl.pallas_call(
    _comb_body,
    grid=(_G,),
    in_specs=[_p0_spec(), _p1_spec(), _p0_spec(), _p0_spec()],
    out_specs=[_p0_spec(), _p0_spec()],
    out_shape=[jax.ShapeDtypeStruct((_NPAD, _D), jnp.float32),
               jax.ShapeDtypeStruct((_NPAD, _D), jnp.float32)],
)


def _final_body(p0, p1, inv, osum, x, fin):
    nv = (p0[...] + p1[...]) * inv[...]
    fin[...] = _ALPHA * x[...] + ((1.0 - _ALPHA) / _STEPS) * (osum[...] + nv)


_final_kernel = pl.pallas_call(
    _final_body,
    grid=(_G,),
    in_specs=[_p0_spec(), _p1_spec(), _p0_spec(), _p0_spec(), _p0_spec()],
    out_specs=_p0_spec(),
    out_shape=jax.ShapeDtypeStruct((_NPAD, _D), jnp.float32),
)


@jax.jit
def kernel(graph_or_x, edge_index):
    x = graph_or_x
    xp = jnp.pad(x, ((0, _NPAD - _N), (0, 0)))
    pad = jnp.full((_EPAD - _E,), _NPAD - 1, jnp.int32)
    srcr = jnp.concatenate([edge_index[0], pad]).reshape(_NW, _NCH, _CHUNK)
    dstr = jnp.concatenate([edge_index[1], pad]).reshape(_NW, _NCH, _CHUNK)
    zeros = jnp.zeros((_ROWS_PER_TILE, _D), jnp.float32)
    ones = jnp.ones((_CHUNK, _D), jnp.float32)

    degp = _deg_kernel(dstr, ones, zeros)
    inv = _inv_kernel(degp, degp)

    cur = xp
    osum = jnp.zeros((_NPAD, _D), jnp.float32)
    for t in range(_STEPS):
        part = _acc_kernel(cur, srcr, dstr, zeros)
        if t < _STEPS - 1:
            cur, osum = _comb_kernel(part, part, inv, osum)
        else:
            fin = _final_kernel(part, part, inv, osum, xp)
    return fin[:_N]


# speedup vs baseline: 2.8341x; 1.0846x over previous
import functools

import jax
import jax.numpy as jnp
from jax import lax
from jax.experimental import pallas as pl
from jax.experimental.pallas import tpu as pltpu
from jax.experimental.pallas import tpu_sc as plsc

_N = 10000
_E = 320000
_D = 128
_STEPS = 10
_ALPHA = 0.1

_NC = 2
_NS = 16
_NW = _NC * _NS
_CHUNK = 128
_NPAD = 10240
_ROWS_PER_TILE = _NPAD // _NS
_EPW = 10240
_NCH = _EPW // _CHUNK
_NHALF = _NCH // 2
_EPAD = _NW * _EPW

_mesh = plsc.VectorSubcoreMesh(core_axis_name="c", subcore_axis_name="s")


def _acc_body(cur_hbm, srcr, dstr, zeros_hbm, part_hbm,
              src_v, dst_v, rows_v, g0, g1, s0, s1, acc_sh):
    rows = (rows_v.at[pl.ds(0, _CHUNK)], rows_v.at[pl.ds(_CHUNK, _CHUNK)])
    gsems = (g0, g1)
    ssems = (s0, s1)
    c = lax.axis_index("c")
    s = lax.axis_index("s")
    wid = c * _NS + s
    pltpu.sync_copy(srcr.at[wid], src_v)
    pltpu.sync_copy(dstr.at[wid, pl.ds(0, _NHALF)], dst_v)
    pltpu.sync_copy(zeros_hbm, acc_sh.at[pl.ds(s * _ROWS_PER_TILE, _ROWS_PER_TILE)])
    plsc.subcore_barrier()

    def gather(j, b):
        pltpu.async_copy(cur_hbm.at[src_v.at[j]], rows[b], gsems[b])

    def gather_wait(j, b):
        pltpu.make_async_copy(cur_hbm.at[src_v.at[j]], rows[b],
                              gsems[b]).wait()

    def scatter(jl, b):
        pltpu.async_copy(rows[b], acc_sh.at[dst_v.at[jl]], ssems[b],
                         add=True)

    def scatter_wait(jl, b):
        pltpu.make_async_copy(rows[b], acc_sh.at[dst_v.at[jl]],
                              ssems[b]).wait()

    gather(0, 0)
    for h in range(2):
        base = h * _NHALF
        gather_wait(base, 0)
        scatter(0, 0)
        gather(base + 1, 1)

        def body(g, carry, base=base):
            for b in (1, 0):
                jl = 2 * g + 1 + (1 - b)
                j = base + jl
                gather_wait(j, b)
                scatter(jl, b)
                scatter_wait(jl - 1, 1 - b)
                gather(j + 1, 1 - b)
            return carry

        lax.fori_loop(0, (_NHALF - 2) // 2, body, None)
        gather_wait(base + _NHALF - 1, 1)
        scatter(_NHALF - 1, 1)
        scatter_wait(_NHALF - 2, 0)
        if h == 0:
            gather(base + _NHALF, 0)
        scatter_wait(_NHALF - 1, 1)
        if h == 0:
            pltpu.sync_copy(dstr.at[wid, pl.ds(_NHALF, _NHALF)], dst_v)

    plsc.subcore_barrier()
    base = c * _NPAD + s * _ROWS_PER_TILE
    pltpu.sync_copy(acc_sh.at[pl.ds(s * _ROWS_PER_TILE, _ROWS_PER_TILE)],
                    part_hbm.at[pl.ds(base, _ROWS_PER_TILE)])


_acc_kernel = functools.partial(
    pl.kernel,
    out_type=jax.ShapeDtypeStruct((_NC * _NPAD, _D), jnp.float32),
    mesh=_mesh,
    scratch_types=[
        pltpu.VMEM((_NCH, _CHUNK), jnp.int32),
        pltpu.VMEM((_NHALF, _CHUNK), jnp.int32),
        pltpu.VMEM((2 * _CHUNK, _D), jnp.float32),
        pltpu.SemaphoreType.DMA,
        pltpu.SemaphoreType.DMA,
        pltpu.SemaphoreType.DMA,
        pltpu.SemaphoreType.DMA,
        pltpu.VMEM_SHARED((_NPAD, _D), jnp.float32),
    ],
)(_acc_body)


def _deg_body(dstr, ones_hbm, zeros_hbm, part_hbm,
              dst_v, ones_v, s0, s1, acc_sh):
    ssems = (s0, s1)
    c = lax.axis_index("c")
    s = lax.axis_index("s")
    wid = c * _NS + s
    pltpu.sync_copy(dstr.at[wid], dst_v)
    pltpu.sync_copy(ones_hbm, ones_v)
    pltpu.sync_copy(zeros_hbm, acc_sh.at[pl.ds(s * _ROWS_PER_TILE, _ROWS_PER_TILE)])
    plsc.subcore_barrier()

    def scatter(j, b):
        pltpu.async_copy(ones_v, acc_sh.at[dst_v.at[j]], ssems[b], add=True)

    def scatter_wait(j, b):
        pltpu.make_async_copy(ones_v, acc_sh.at[dst_v.at[j]],
                              ssems[b]).wait()

    scatter(0, 0)
    scatter(1, 1)

    def body(g, carry):
        for b in (0, 1):
            j = 2 * g + 2 + b
            scatter_wait(j - 2, b)
            scatter(j, b)
        return carry

    lax.fori_loop(0, (_NCH - 2) // 2, body, None)
    scatter_wait(_NCH - 2, 0)
    scatter_wait(_NCH - 1, 1)
    plsc.subcore_barrier()
    base = c * _NPAD + s * _ROWS_PER_TILE
    pltpu.sync_copy(acc_sh.at[pl.ds(s * _ROWS_PER_TILE, _ROWS_PER_TILE)],
                    part_hbm.at[pl.ds(base, _ROWS_PER_TILE)])


_deg_kernel = functools.partial(
    pl.kernel,
    out_type=jax.ShapeDtypeStruct((_NC * _NPAD, _D), jnp.float32),
    mesh=_mesh,
    scratch_types=[
        pltpu.VMEM((_NCH, _CHUNK), jnp.int32),
        pltpu.VMEM((_CHUNK, _D), jnp.float32),
        pltpu.SemaphoreType.DMA,
        pltpu.SemaphoreType.DMA,
        pltpu.VMEM_SHARED((_NPAD, _D), jnp.float32),
    ],
)(_deg_body)


_BLK = 1280
_G = _NPAD // _BLK


def _p0_spec():
    return pl.BlockSpec((_BLK, _D), lambda i: (i, 0))


def _p1_spec():
    return pl.BlockSpec((_BLK, _D), lambda i: (i + _G, 0))


def _inv_body(d0, d1, inv):
    inv[...] = 1.0 / jnp.maximum(d0[...] + d1[...], 1.0)


_inv_kernel = pl.pallas_call(
    _inv_body,
    grid=(_G,),
    in_specs=[_p0_spec(), _p1_spec()],
    out_specs=_p0_spec(),
    out_shape=jax.ShapeDtypeStruct((_NPAD, _D), jnp.float32),
)


def _comb_body(p0, p1, inv, osum, nxt, osum_o):
    nv = (p0[...] + p1[...]) * inv[...]
    nxt[...] = nv
    osum_o[...] = osum[...] + nv


_comb_kernel = pl.pallas_call(
    _comb_body,
    grid=(_G,),
    in_specs=[_p0_spec(), _p1_spec(), _p0_spec(), _p0_spec()],
    out_specs=[_p0_spec(), _p0_spec()],
    out_shape=[jax.ShapeDtypeStruct((_NPAD, _D), jnp.float32),
               jax.ShapeDtypeStruct((_NPAD, _D), jnp.float32)],
)


def _final_body(p0, p1, inv, osum, x, fin):
    nv = (p0[...] + p1[...]) * inv[...]
    fin[...] = _ALPHA * x[...] + ((1.0 - _ALPHA) / _STEPS) * (osum[...] + nv)


_final_kernel = pl.pallas_call(
    _final_body,
    grid=(_G,),
    in_specs=[_p0_spec(), _p1_spec(), _p0_spec(), _p0_spec(), _p0_spec()],
    out_specs=_p0_spec(),
    out_shape=jax.ShapeDtypeStruct((_NPAD, _D), jnp.float32),
)


@jax.jit
def kernel(graph_or_x, edge_index):
    x = graph_or_x
    xp = jnp.pad(x, ((0, _NPAD - _N), (0, 0)))
    pad = jnp.full((_EPAD - _E,), _NPAD - 1, jnp.int32)
    srcr = jnp.concatenate([edge_index[0], pad]).reshape(_NW, _NCH, _CHUNK)
    dstr = jnp.concatenate([edge_index[1], pad]).reshape(_NW, _NCH, _CHUNK)
    zeros = jnp.zeros((_ROWS_PER_TILE, _D), jnp.float32)
    ones = jnp.ones((_CHUNK, _D), jnp.float32)

    degp = _deg_kernel(dstr, ones, zeros)
    inv = _inv_kernel(degp, degp)

    cur = xp
    osum = jnp.zeros((_NPAD, _D), jnp.float32)
    for t in range(_STEPS):
        part = _acc_kernel(cur, srcr, dstr, zeros)
        if t < _STEPS - 1:
            cur, osum = _comb_kernel(part, part, inv, osum)
        else:
            fin = _final_kernel(part, part, inv, osum, xp)
    return fin[:_N]


# speedup vs baseline: 3.1572x; 1.1140x over previous
import functools

import jax
import jax.numpy as jnp
from jax import lax
from jax.experimental import pallas as pl
from jax.experimental.pallas import tpu as pltpu
from jax.experimental.pallas import tpu_sc as plsc

_N = 10000
_E = 320000
_D = 128
_STEPS = 10
_ALPHA = 0.1

_NC = 2
_NS = 16
_NW = _NC * _NS
_CHUNK = 128
_NPAD = 10240
_ROWS_PER_TILE = _NPAD // _NS
_EPW = 10240
_NCH = _EPW // _CHUNK
_NHALF = _NCH // 2
_EPAD = _NW * _EPW

_mesh = plsc.VectorSubcoreMesh(core_axis_name="c", subcore_axis_name="s")


def _acc_body(cur_hbm, srcr, dstr, zeros_hbm, part_hbm,
              src_v, dst_v, rows_v, g0, g1, s0, s1, acc_sh):
    rows = (rows_v.at[pl.ds(0, _CHUNK)], rows_v.at[pl.ds(_CHUNK, _CHUNK)])
    gsems = (g0, g1)
    ssems = (s0, s1)
    c = lax.axis_index("c")
    s = lax.axis_index("s")
    wid = c * _NS + s
    pltpu.sync_copy(srcr.at[wid], src_v)
    pltpu.sync_copy(dstr.at[wid, pl.ds(0, _NHALF)], dst_v)
    pltpu.sync_copy(zeros_hbm, acc_sh.at[pl.ds(s * _ROWS_PER_TILE, _ROWS_PER_TILE)])
    plsc.subcore_barrier()

    def gather(j, b):
        pltpu.async_copy(cur_hbm.at[src_v.at[j]], rows[b], gsems[b])

    def gather_wait(j, b):
        pltpu.make_async_copy(cur_hbm.at[src_v.at[j]], rows[b],
                              gsems[b]).wait()

    def scatter(jl, b):
        pltpu.async_copy(rows[b], acc_sh.at[dst_v.at[jl]], ssems[b],
                         add=True)

    def scatter_wait(jl, b):
        pltpu.make_async_copy(rows[b], acc_sh.at[dst_v.at[jl]],
                              ssems[b]).wait()

    gather(0, 0)
    for h in range(2):
        base = h * _NHALF
        gather_wait(base, 0)
        scatter(0, 0)
        gather(base + 1, 1)

        def body(g, carry, base=base):
            for b in (1, 0):
                jl = 2 * g + 1 + (1 - b)
                j = base + jl
                gather_wait(j, b)
                scatter(jl, b)
                scatter_wait(jl - 1, 1 - b)
                gather(j + 1, 1 - b)
            return carry

        lax.fori_loop(0, (_NHALF - 2) // 2, body, None)
        gather_wait(base + _NHALF - 1, 1)
        scatter(_NHALF - 1, 1)
        scatter_wait(_NHALF - 2, 0)
        if h == 0:
            gather(base + _NHALF, 0)
        scatter_wait(_NHALF - 1, 1)
        if h == 0:
            pltpu.sync_copy(dstr.at[wid, pl.ds(_NHALF, _NHALF)], dst_v)

    plsc.subcore_barrier()
    base = c * _NPAD + s * _ROWS_PER_TILE
    pltpu.sync_copy(acc_sh.at[pl.ds(s * _ROWS_PER_TILE, _ROWS_PER_TILE)],
                    part_hbm.at[pl.ds(base, _ROWS_PER_TILE)])


_acc_kernel = functools.partial(
    pl.kernel,
    out_type=jax.ShapeDtypeStruct((_NC * _NPAD, _D), jnp.float32),
    mesh=_mesh,
    scratch_types=[
        pltpu.VMEM((_NCH, _CHUNK), jnp.int32),
        pltpu.VMEM((_NHALF, _CHUNK), jnp.int32),
        pltpu.VMEM((2 * _CHUNK, _D), jnp.float32),
        pltpu.SemaphoreType.DMA,
        pltpu.SemaphoreType.DMA,
        pltpu.SemaphoreType.DMA,
        pltpu.SemaphoreType.DMA,
        pltpu.VMEM_SHARED((_NPAD, _D), jnp.float32),
    ],
)(_acc_body)


def _deg_body(dstr, ones_hbm, zeros_hbm, part_hbm,
              dst_v, ones_v, s0, s1, acc_sh):
    ssems = (s0, s1)
    c = lax.axis_index("c")
    s = lax.axis_index("s")
    wid = c * _NS + s
    pltpu.sync_copy(dstr.at[wid], dst_v)
    pltpu.sync_copy(ones_hbm, ones_v)
    pltpu.sync_copy(zeros_hbm, acc_sh.at[pl.ds(s * _ROWS_PER_TILE, _ROWS_PER_TILE)])
    plsc.subcore_barrier()

    def scatter(j, b):
        pltpu.async_copy(ones_v, acc_sh.at[dst_v.at[j]], ssems[b], add=True)

    def scatter_wait(j, b):
        pltpu.make_async_copy(ones_v, acc_sh.at[dst_v.at[j]],
                              ssems[b]).wait()

    scatter(0, 0)
    scatter(1, 1)

    def body(g, carry):
        for b in (0, 1):
            j = 2 * g + 2 + b
            scatter_wait(j - 2, b)
            scatter(j, b)
        return carry

    lax.fori_loop(0, (_NCH - 2) // 2, body, None)
    scatter_wait(_NCH - 2, 0)
    scatter_wait(_NCH - 1, 1)
    plsc.subcore_barrier()
    base = c * _NPAD + s * _ROWS_PER_TILE
    pltpu.sync_copy(acc_sh.at[pl.ds(s * _ROWS_PER_TILE, _ROWS_PER_TILE)],
                    part_hbm.at[pl.ds(base, _ROWS_PER_TILE)])


_DW = 16


_deg_kernel = functools.partial(
    pl.kernel,
    out_type=jax.ShapeDtypeStruct((_NC * _NPAD, _DW), jnp.float32),
    mesh=_mesh,
    scratch_types=[
        pltpu.VMEM((_NCH, _CHUNK), jnp.int32),
        pltpu.VMEM((_CHUNK, _DW), jnp.float32),
        pltpu.SemaphoreType.DMA,
        pltpu.SemaphoreType.DMA,
        pltpu.VMEM_SHARED((_NPAD, _DW), jnp.float32),
    ],
)(_deg_body)


_BLK = 1280
_G = _NPAD // _BLK


def _p0_spec():
    return pl.BlockSpec((_BLK, _D), lambda i: (i, 0))


def _p1_spec():
    return pl.BlockSpec((_BLK, _D), lambda i: (i + _G, 0))


def _inv_body(d0, d1, inv):
    d = jnp.maximum(d0[...] + d1[...], 1.0)
    inv[...] = 1.0 / jnp.broadcast_to(d[:, :1], (_BLK, _D))


_inv_kernel = pl.pallas_call(
    _inv_body,
    grid=(_G,),
    in_specs=[pl.BlockSpec((_BLK, 16), lambda i: (i, 0)),
              pl.BlockSpec((_BLK, 16), lambda i: (i + _G, 0))],
    out_specs=_p0_spec(),
    out_shape=jax.ShapeDtypeStruct((_NPAD, _D), jnp.float32),
)


def _comb_body(p0, p1, inv, osum, nxt, osum_o):
    nv = (p0[...] + p1[...]) * inv[...]
    nxt[...] = nv
    osum_o[...] = osum[...] + nv


_comb_kernel = pl.pallas_call(
    _comb_body,
    grid=(_G,),
    in_specs=[_p0_spec(), _p1_spec(), _p0_spec(), _p0_spec()],
    out_specs=[_p0_spec(), _p0_spec()],
    out_shape=[jax.ShapeDtypeStruct((_NPAD, _D), jnp.float32),
               jax.ShapeDtypeStruct((_NPAD, _D), jnp.float32)],
)


def _final_body(p0, p1, inv, osum, x, fin):
    nv = (p0[...] + p1[...]) * inv[...]
    fin[...] = _ALPHA * x[...] + ((1.0 - _ALPHA) / _STEPS) * (osum[...] + nv)


_final_kernel = pl.pallas_call(
    _final_body,
    grid=(_G,),
    in_specs=[_p0_spec(), _p1_spec(), _p0_spec(), _p0_spec(), _p0_spec()],
    out_specs=_p0_spec(),
    out_shape=jax.ShapeDtypeStruct((_NPAD, _D), jnp.float32),
)


@jax.jit
def kernel(graph_or_x, edge_index):
    x = graph_or_x
    xp = jnp.pad(x, ((0, _NPAD - _N), (0, 0)))
    pad = jnp.full((_EPAD - _E,), _NPAD - 1, jnp.int32)
    srcr = jnp.concatenate([edge_index[0], pad]).reshape(_NW, _NCH, _CHUNK)
    dstr = jnp.concatenate([edge_index[1], pad]).reshape(_NW, _NCH, _CHUNK)
    zeros = jnp.zeros((_ROWS_PER_TILE, _D), jnp.float32)
    ones16 = jnp.ones((_CHUNK, 16), jnp.float32)
    zeros16 = jnp.zeros((_ROWS_PER_TILE, 16), jnp.float32)

    degp = _deg_kernel(dstr, ones16, zeros16)
    inv = _inv_kernel(degp, degp)

    cur = xp
    osum = jnp.zeros((_NPAD, _D), jnp.float32)
    for t in range(_STEPS):
        part = _acc_kernel(cur, srcr, dstr, zeros)
        if t < _STEPS - 1:
            cur, osum = _comb_kernel(part, part, inv, osum)
        else:
            fin = _final_kernel(part, part, inv, osum, xp)
    return fin[:_N]
